# Initial kernel scaffold; baseline (speedup 1.0000x reference)
import jax
import jax.numpy as jnp
from jax.experimental import pallas as pl


def kernel(sequence_data, is_rnn, v_len, Wh, bh, Ww, bw):
    raise NotImplementedError("write your pallas kernel here")


# speedup vs baseline: 8.6089x; 8.6089x over previous
import functools
import jax
import jax.numpy as jnp
from jax import lax
from jax.experimental import pallas as pl
from jax.experimental.pallas import tpu as pltpu
from jax.experimental.pallas import tpu_sc as plsc

L, B, D, T = 4096, 16, 64, 32
HC = L // 2
TB = 512
NSUB = HC // TB
NG = TB // 16


def _body(x_hbm, ir_hbm, vl_hbm, wh_hbm, bh_hbm, ww_hbm, bw_hbm, out_hbm,
          xbuf, mbuf, attv, pbuf, whb, bhb, wwb, bwb, hwb, vlb,
          shatt, shf32, shhw):
    c = lax.axis_index("c")
    s = lax.axis_index("s")
    si = s // 2
    ch = s % 2
    i = c * 8 + si
    f32 = jnp.float32

    pltpu.sync_copy(vl_hbm, vlb)
    pltpu.sync_copy(ir_hbm.at[i, pl.ds(ch * HC, HC)], mbuf)
    mv0 = vlb[pl.ds(0, 16)]
    m = mv0[i] if isinstance(i, int) else jnp.sum(
        jnp.where(lax.iota(jnp.int32, 16) == i, mv0, 0))

    zeros16i = jnp.zeros((16,), jnp.int32)

    def _cnt(j, acc):
        return acc + mbuf[pl.ds(j * 16, 16)]

    cvec = lax.fori_loop(0, HC // 16, _cnt, zeros16i)
    my_cnt = jnp.sum(cvec)
    prefix = jnp.where(ch == 0, 0, m - my_cnt)

    big = jnp.maximum(m, T)
    small = jnp.clip(jnp.minimum(m, T), 1, T)
    gs = big // small
    rem = big % small
    thr = rem * (gs + 1)
    mmax = jnp.maximum(m - 1, 0)

    def bucket(p):
        bb = jnp.where(p < thr, p // (gs + 1), rem + (p - thr) // gs)
        return jnp.clip(bb, 0, T - 1)

    zeros16f = jnp.zeros((16,), f32)

    def _z(j, _):
        attv[pl.ds(j * 16, 16)] = zeros16f
        pbuf[pl.ds(j * 16, 16)] = zeros16f
        return 0

    lax.fori_loop(0, (T * D) // 16, _z, 0)

    is_lt = m < T

    @pl.when((ch == 0) & is_lt)
    def _zero_first32():
        pltpu.sync_copy(pbuf, shf32.at[si])

    @pl.when(ch == 0)
    def _stage_w():
        pltpu.sync_copy(wh_hbm, whb)
        pltpu.sync_copy(bh_hbm, bhb)
        pltpu.sync_copy(ww_hbm, wwb)
        pltpu.sync_copy(bw_hbm, bwb)

    plsc.subcore_barrier()

    def _sub1(sb, cnt_in):
        t0 = ch * HC + sb * TB
        pltpu.sync_copy(x_hbm.at[i, pl.ds(t0, TB)], xbuf)

        def grp(gi, cnt):
            base = sb * TB + gi * 16
            mvec = mbuf[pl.ds(base, 16)]
            cums = plsc.cumsum(mvec)
            pvec = jnp.clip(prefix + cnt + cums - 1, 0, mmax)
            bvec = bucket(pvec)
            mff = mvec.astype(f32)
            for j in range(16):
                bj = bvec[j]
                mj = mff[j]
                for g in range(4):
                    off = bj * D + g * 16
                    attv[pl.ds(off, 16)] = (
                        attv[pl.ds(off, 16)]
                        + xbuf[gi * 16 + j, pl.ds(g * 16, 16)] * mj)
            return cnt + cums[15]

        cnt_out = lax.fori_loop(0, NG, grp, cnt_in)

        @pl.when(is_lt)
        def _first32():
            def grp2(gi, cnt):
                base = sb * TB + gi * 16
                mvec = mbuf[pl.ds(base, 16)]
                cums = plsc.cumsum(mvec)
                pvec = prefix + cnt + cums - 1
                for j in range(16):
                    pj = pvec[j]

                    @pl.when((mvec[j] > 0) & (pj < T))
                    def _():
                        pltpu.sync_copy(xbuf.at[gi * 16 + j],
                                        shf32.at[si, pl.ds(pj * D, D)])

                return cnt + cums[15]

            lax.fori_loop(0, NG, grp2, cnt_in)

        return cnt_out

    lax.fori_loop(0, NSUB, _sub1, jnp.int32(0))

    pltpu.sync_copy(attv, shatt.at[s])
    plsc.subcore_barrier()

    @pl.when(ch == 0)
    def _phase2():
        pltpu.sync_copy(shatt.at[s + 1], pbuf)

        def addp(j, _):
            attv[pl.ds(j * 16, 16)] = (attv[pl.ds(j * 16, 16)]
                                       + pbuf[pl.ds(j * 16, 16)])
            return 0

        lax.fori_loop(0, (T * D) // 16, addp, 0)

        @pl.when(m >= T)
        def _gt_mean():
            def rowdiv(j, _):
                cj = (gs + jnp.where(j < rem, 1, 0)).astype(f32)
                rcv = jnp.full((16,), 1.0, f32) / cj
                for g in range(4):
                    off = j * D + g * 16
                    attv[pl.ds(off, 16)] = attv[pl.ds(off, 16)] * rcv
                return 0

            lax.fori_loop(0, T, rowdiv, 0)

        @pl.when(is_lt)
        def _lt_expand():
            pltpu.sync_copy(shf32.at[si], pbuf)

            def rowcp(j, _):
                sidx = jnp.where(j < thr, j // (gs + 1),
                                 rem + (j - thr) // gs)
                sidx = jnp.clip(sidx, 0, T - 1)
                for g in range(4):
                    attv[pl.ds(j * D + g * 16, 16)] = (
                        pbuf[pl.ds(sidx * D + g * 16, 16)])
                return 0

            lax.fori_loop(0, T, rowcp, 0)

        iota16 = lax.iota(jnp.int32, 16)

        ad = [zeros16f, zeros16f]
        for j in range(T):
            acc = attv[pl.ds(j * D, 16)]
            for g in range(1, 4):
                acc = acc + attv[pl.ds(j * D + g * 16, 16)]
            rsum = jnp.sum(acc) * (1.0 / D)
            gslot = j // 16
            ad[gslot] = ad[gslot] + jnp.where(iota16 == (j % 16), rsum, 0.0)

        def colacc(j, carry):
            return tuple(carry[g] + attv[pl.ds(j * D + g * 16, 16)]
                         for g in range(4))

        at4 = [v * (1.0 / T) for v in
               lax.fori_loop(0, T, colacc, (zeros16f,) * 4)]

        hv = []
        for g in range(2):
            zacc = zeros16f
            for u in range(T):
                col = plsc.load_gather(whb, [iota16 * T + (g * 16 * T + u)])
                zacc = zacc + col * ad[u // 16][u % 16]
            z = zacc + bhb[pl.ds(g * 16, 16)]
            hv.append(1.0 / (1.0 + jnp.exp(-z)))

        for g in range(4):
            zacc = zeros16f
            for e in range(D):
                col = plsc.load_gather(wwb, [iota16 * D + (g * 16 * D + e)])
                zacc = zacc + col * at4[e // 16][e % 16]
            z = zacc + bwb[pl.ds(g * 16, 16)]
            hwb[pl.ds(T + g * 16, 16)] = 1.0 / (1.0 + jnp.exp(-z))

        @pl.when(m >= T)
        def _hc_gt():
            for g in range(2):
                hwb[pl.ds(g * 16, 16)] = hv[g]

        @pl.when(is_lt)
        def _hc_lt():
            hacc = [zeros16f, zeros16f]
            hcnt = [zeros16f, zeros16f]
            for j in range(T):
                sidx = jnp.where(j < thr, j // (gs + 1),
                                 rem + (j - thr) // gs)
                sidx = jnp.clip(sidx, 0, T - 1)
                hj = hv[j // 16][j % 16]
                for g in range(2):
                    hit = (iota16 + g * 16) == sidx
                    hacc[g] = hacc[g] + jnp.where(hit, hj, 0.0)
                    hcnt[g] = hcnt[g] + jnp.where(hit, 1.0, 0.0)
            for g in range(2):
                hwb[pl.ds(g * 16, 16)] = (hacc[g]
                                          / jnp.maximum(hcnt[g], 1.0))

        pltpu.sync_copy(hwb, shhw.at[si])

    plsc.subcore_barrier()
    pltpu.sync_copy(shhw.at[si], hwb)

    whalf = [hwb[pl.ds(T + g * 16, 16)] * 0.5 for g in range(4)]
    m_ge = m >= T

    def _sub3(sb, cnt_in):
        t0 = ch * HC + sb * TB
        pltpu.sync_copy(x_hbm.at[i, pl.ds(t0, TB)], xbuf)

        def grp(gi, cnt):
            base = sb * TB + gi * 16
            mvec = mbuf[pl.ds(base, 16)]
            cums = plsc.cumsum(mvec)
            pvec = jnp.clip(prefix + cnt + cums - 1, 0, mmax)
            idxv = jnp.where(m_ge, bucket(pvec), pvec)
            hs2 = plsc.load_gather(hwb, [idxv]) * 0.5
            mff = mvec.astype(f32)
            omv = 1.0 - mff
            for j in range(16):
                hj = hs2[j]
                mj = mff[j]
                oj = omv[j]
                for g in range(4):
                    scale = (whalf[g] + hj) * mj + oj
                    xbuf[gi * 16 + j, pl.ds(g * 16, 16)] = (
                        xbuf[gi * 16 + j, pl.ds(g * 16, 16)] * scale)
            return cnt + cums[15]

        cnt_out = lax.fori_loop(0, NG, grp, cnt_in)
        pltpu.sync_copy(xbuf, out_hbm.at[i, pl.ds(t0, TB)])
        return cnt_out

    lax.fori_loop(0, NSUB, _sub3, jnp.int32(0))


@jax.jit
def _fwd(x, ir, vl, Wh, bh, Ww, bw):
    mesh = plsc.VectorSubcoreMesh(core_axis_name="c", subcore_axis_name="s")
    f32 = jnp.float32
    kfn = functools.partial(
        pl.kernel,
        out_type=jax.ShapeDtypeStruct((B, L, D), f32),
        mesh=mesh,
        compiler_params=pltpu.CompilerParams(use_tc_tiling_on_sc=False,
                                             needs_layout_passes=False),
        scratch_types=[
            pltpu.VMEM((TB, D), f32),
            pltpu.VMEM((HC,), jnp.int32),
            pltpu.VMEM((T * D,), f32),
            pltpu.VMEM((T * D,), f32),
            pltpu.VMEM((T * T,), f32),
            pltpu.VMEM((T,), f32),
            pltpu.VMEM((D * D,), f32),
            pltpu.VMEM((D,), f32),
            pltpu.VMEM((128,), f32),
            pltpu.VMEM((B,), jnp.int32),
            pltpu.VMEM_SHARED((16, T * D), f32),
            pltpu.VMEM_SHARED((8, T * D), f32),
            pltpu.VMEM_SHARED((8, 128), f32),
        ],
    )(_body)
    xT = jnp.transpose(x, (1, 0, 2))
    irT = jnp.transpose(ir, (1, 0))
    outT = kfn(xT, irT, vl, Wh.reshape(T * T), bh, Ww.reshape(D * D), bw)
    return jnp.transpose(outT, (1, 0, 2))


def kernel(sequence_data, is_rnn, v_len, Wh, bh, Ww, bw):
    seq_out = _fwd(sequence_data, is_rnn.astype(jnp.int32),
                   v_len.astype(jnp.int32), Wh, bh, Ww, bw)
    return (seq_out, is_rnn)


# speedup vs baseline: 9.8387x; 1.1429x over previous
import functools
import jax
import jax.numpy as jnp
from jax import lax
from jax.experimental import pallas as pl
from jax.experimental.pallas import tpu as pltpu
from jax.experimental.pallas import tpu_sc as plsc

L, B, D, T = 4096, 16, 64, 32
HC = L // 2
TB = 256
NSUB = HC // TB
NG = TB // 16
NQ = TB // 128
NT = NSUB // 2


def _body(x_hbm, ir_hbm, vl_hbm, wh_hbm, bh_hbm, ww_hbm, bw_hbm, out_hbm,
          ib0, ib1, ob0, ob1, idxc, mbuf, attv, pbuf,
          whb, bhb, wwb, bwb, hwb, vlb,
          gsm0, gsm1, ssm0, ssm1, shatt, shf32, shhw):
    c = lax.axis_index("c")
    s = lax.axis_index("s")
    si = s // 2
    ch = s % 2
    i = c * 8 + si
    f32 = jnp.float32

    pltpu.sync_copy(vl_hbm, vlb)
    pltpu.sync_copy(ir_hbm.at[i, pl.ds(ch * HC, HC)], mbuf)

    iota16b = lax.iota(jnp.int32, 16) * B

    def _bld(q, _):
        for v in range(8):
            basei = (ch * HC + q * 128 + v * 16) * B + i
            idxc[q, pl.ds(v * 16, 16)] = iota16b + basei
        return 0

    lax.fori_loop(0, HC // 128, _bld, 0)

    ibufs = (ib0, ib1)
    obufs = (ob0, ob1)
    gsems = (gsm0, gsm1)
    ssems = (ssm0, ssm1)

    def issue_gather(sb, buf, sem):
        for qq in range(NQ):
            pltpu.async_copy(x_hbm.at[idxc.at[sb * NQ + qq]],
                             buf.at[pl.ds(qq * 128, 128)], sem)

    def wait_gather(sb, buf, sem):
        for qq in range(NQ):
            pltpu.make_async_copy(x_hbm.at[idxc.at[sb * NQ + qq]],
                                  buf.at[pl.ds(qq * 128, 128)], sem).wait()

    def issue_scatter(sb, buf, sem):
        for qq in range(NQ):
            pltpu.async_copy(buf.at[pl.ds(qq * 128, 128)],
                             out_hbm.at[idxc.at[sb * NQ + qq]], sem)

    def wait_scatter(sb, buf, sem):
        for qq in range(NQ):
            pltpu.make_async_copy(buf.at[pl.ds(qq * 128, 128)],
                                  out_hbm.at[idxc.at[sb * NQ + qq]],
                                  sem).wait()

    mv0 = vlb[pl.ds(0, 16)]
    m = jnp.sum(jnp.where(lax.iota(jnp.int32, 16) == i, mv0, 0))

    zeros16i = jnp.zeros((16,), jnp.int32)

    def _cnt(j, acc):
        return acc + mbuf[pl.ds(j * 16, 16)]

    cvec = lax.fori_loop(0, HC // 16, _cnt, zeros16i)
    my_cnt = jnp.sum(cvec)
    prefix = jnp.where(ch == 0, 0, m - my_cnt)

    big = jnp.maximum(m, T)
    small = jnp.clip(jnp.minimum(m, T), 1, T)
    gs = big // small
    rem = big % small
    thr = rem * (gs + 1)
    mmax = jnp.maximum(m - 1, 0)

    def bucket(p):
        bb = jnp.where(p < thr, p // (gs + 1), rem + (p - thr) // gs)
        return jnp.clip(bb, 0, T - 1)

    zeros16f = jnp.zeros((16,), f32)

    def _z(j, _):
        attv[pl.ds(j * 16, 16)] = zeros16f
        pbuf[pl.ds(j * 16, 16)] = zeros16f
        return 0

    lax.fori_loop(0, (T * D) // 16, _z, 0)

    is_lt = m < T

    @pl.when((ch == 0) & is_lt)
    def _zero_first32():
        pltpu.sync_copy(pbuf, shf32.at[si])

    @pl.when(ch == 0)
    def _stage_w():
        pltpu.sync_copy(wh_hbm, whb)
        pltpu.sync_copy(bh_hbm, bhb)
        pltpu.sync_copy(ww_hbm, wwb)
        pltpu.sync_copy(bw_hbm, bwb)

    plsc.subcore_barrier()

    def _p1_compute(sb, cnt_in, xcur):
        def grp(gi, cnt):
            base = sb * TB + gi * 16
            mvec = mbuf[pl.ds(base, 16)]
            cums = plsc.cumsum(mvec)
            pvec = jnp.clip(prefix + cnt + cums - 1, 0, mmax)
            bvec = bucket(pvec)
            mff = mvec.astype(f32)
            for j in range(16):
                bj = bvec[j]
                mj = mff[j]
                for g in range(4):
                    off = bj * D + g * 16
                    attv[pl.ds(off, 16)] = (
                        attv[pl.ds(off, 16)]
                        + xcur[gi * 16 + j, pl.ds(g * 16, 16)] * mj)
            return cnt + cums[15]

        cnt_out = lax.fori_loop(0, NG, grp, cnt_in)

        @pl.when(is_lt)
        def _first32():
            def grp2(gi, cnt):
                base = sb * TB + gi * 16
                mvec = mbuf[pl.ds(base, 16)]
                cums = plsc.cumsum(mvec)
                pvec = prefix + cnt + cums - 1
                for j in range(16):
                    pj = pvec[j]

                    @pl.when((mvec[j] > 0) & (pj < T))
                    def _():
                        pltpu.sync_copy(xcur.at[gi * 16 + j],
                                        shf32.at[si, pl.ds(pj * D, D)])

                return cnt + cums[15]

            lax.fori_loop(0, NG, grp2, cnt_in)

        return cnt_out

    issue_gather(0, ibufs[0], gsems[0])
    issue_gather(1, ibufs[1], gsems[1])

    def _p1_outer(t2, cnt):
        for b2 in range(2):
            sb = 2 * t2 + b2
            wait_gather(sb, ibufs[b2], gsems[b2])
            cnt = _p1_compute(sb, cnt, ibufs[b2])

            @pl.when(t2 < NT - 1)
            def _():
                issue_gather(sb + 2, ibufs[b2], gsems[b2])

        return cnt

    lax.fori_loop(0, NT, _p1_outer, jnp.int32(0))

    pltpu.sync_copy(attv, shatt.at[s])
    plsc.subcore_barrier()

    @pl.when(ch == 0)
    def _phase2():
        pltpu.sync_copy(shatt.at[s + 1], pbuf)

        def addp(j, _):
            attv[pl.ds(j * 16, 16)] = (attv[pl.ds(j * 16, 16)]
                                       + pbuf[pl.ds(j * 16, 16)])
            return 0

        lax.fori_loop(0, (T * D) // 16, addp, 0)

        @pl.when(m >= T)
        def _gt_mean():
            def rowdiv(j, _):
                cj = (gs + jnp.where(j < rem, 1, 0)).astype(f32)
                rcv = jnp.full((16,), 1.0, f32) / cj
                for g in range(4):
                    off = j * D + g * 16
                    attv[pl.ds(off, 16)] = attv[pl.ds(off, 16)] * rcv
                return 0

            lax.fori_loop(0, T, rowdiv, 0)

        @pl.when(is_lt)
        def _lt_expand():
            pltpu.sync_copy(shf32.at[si], pbuf)

            def rowcp(j, _):
                sidx = jnp.where(j < thr, j // (gs + 1),
                                 rem + (j - thr) // gs)
                sidx = jnp.clip(sidx, 0, T - 1)
                for g in range(4):
                    attv[pl.ds(j * D + g * 16, 16)] = (
                        pbuf[pl.ds(sidx * D + g * 16, 16)])
                return 0

            lax.fori_loop(0, T, rowcp, 0)

        iota16 = lax.iota(jnp.int32, 16)

        ad = [zeros16f, zeros16f]
        for j in range(T):
            acc = attv[pl.ds(j * D, 16)]
            for g in range(1, 4):
                acc = acc + attv[pl.ds(j * D + g * 16, 16)]
            rsum = jnp.sum(acc) * (1.0 / D)
            gslot = j // 16
            ad[gslot] = ad[gslot] + jnp.where(iota16 == (j % 16), rsum, 0.0)

        def colacc(j, carry):
            return tuple(carry[g] + attv[pl.ds(j * D + g * 16, 16)]
                         for g in range(4))

        at4 = [v * (1.0 / T) for v in
               lax.fori_loop(0, T, colacc, (zeros16f,) * 4)]

        hv = []
        for g in range(2):
            zacc = zeros16f
            for u in range(T):
                col = plsc.load_gather(whb, [iota16 * T + (g * 16 * T + u)])
                zacc = zacc + col * ad[u // 16][u % 16]
            z = zacc + bhb[pl.ds(g * 16, 16)]
            hv.append(1.0 / (1.0 + jnp.exp(-z)))

        for g in range(4):
            zacc = zeros16f
            for e in range(D):
                col = plsc.load_gather(wwb, [iota16 * D + (g * 16 * D + e)])
                zacc = zacc + col * at4[e // 16][e % 16]
            z = zacc + bwb[pl.ds(g * 16, 16)]
            hwb[pl.ds(T + g * 16, 16)] = 1.0 / (1.0 + jnp.exp(-z))

        @pl.when(m >= T)
        def _hc_gt():
            for g in range(2):
                hwb[pl.ds(g * 16, 16)] = hv[g]

        @pl.when(is_lt)
        def _hc_lt():
            hacc = [zeros16f, zeros16f]
            hcnt = [zeros16f, zeros16f]
            for j in range(T):
                sidx = jnp.where(j < thr, j // (gs + 1),
                                 rem + (j - thr) // gs)
                sidx = jnp.clip(sidx, 0, T - 1)
                hj = hv[j // 16][j % 16]
                for g in range(2):
                    hit = (iota16 + g * 16) == sidx
                    hacc[g] = hacc[g] + jnp.where(hit, hj, 0.0)
                    hcnt[g] = hcnt[g] + jnp.where(hit, 1.0, 0.0)
            for g in range(2):
                hwb[pl.ds(g * 16, 16)] = (hacc[g]
                                          / jnp.maximum(hcnt[g], 1.0))

        pltpu.sync_copy(hwb, shhw.at[si])

    plsc.subcore_barrier()
    pltpu.sync_copy(shhw.at[si], hwb)

    whalf = [hwb[pl.ds(T + g * 16, 16)] * 0.5 for g in range(4)]
    m_ge = m >= T

    def _p3_compute(sb, cnt_in, xcur, xout):
        def grp(gi, cnt):
            base = sb * TB + gi * 16
            mvec = mbuf[pl.ds(base, 16)]
            cums = plsc.cumsum(mvec)
            pvec = jnp.clip(prefix + cnt + cums - 1, 0, mmax)
            idxv = jnp.where(m_ge, bucket(pvec), pvec)
            hs2 = plsc.load_gather(hwb, [idxv]) * 0.5
            mff = mvec.astype(f32)
            omv = 1.0 - mff
            for j in range(16):
                hj = hs2[j]
                mj = mff[j]
                oj = omv[j]
                for g in range(4):
                    scale = (whalf[g] + hj) * mj + oj
                    xout[gi * 16 + j, pl.ds(g * 16, 16)] = (
                        xcur[gi * 16 + j, pl.ds(g * 16, 16)] * scale)
            return cnt + cums[15]

        return lax.fori_loop(0, NG, grp, cnt_in)

    issue_gather(0, ibufs[0], gsems[0])
    issue_gather(1, ibufs[1], gsems[1])

    def _p3_outer(t2, cnt):
        for b2 in range(2):
            sb = 2 * t2 + b2
            wait_gather(sb, ibufs[b2], gsems[b2])

            @pl.when(t2 > 0)
            def _():
                wait_scatter(sb - 2, obufs[b2], ssems[b2])

            cnt = _p3_compute(sb, cnt, ibufs[b2], obufs[b2])
            issue_scatter(sb, obufs[b2], ssems[b2])

            @pl.when(t2 < NT - 1)
            def _():
                issue_gather(sb + 2, ibufs[b2], gsems[b2])

        return cnt

    lax.fori_loop(0, NT, _p3_outer, jnp.int32(0))
    for b2 in range(2):
        wait_scatter(NSUB - 2 + b2, obufs[b2], ssems[b2])


@jax.jit
def _fwd(x, ir, vl, Wh, bh, Ww, bw):
    mesh = plsc.VectorSubcoreMesh(core_axis_name="c", subcore_axis_name="s")
    f32 = jnp.float32
    kfn = functools.partial(
        pl.kernel,
        out_type=jax.ShapeDtypeStruct((L * B, D), f32),
        mesh=mesh,
        compiler_params=pltpu.CompilerParams(use_tc_tiling_on_sc=False,
                                             needs_layout_passes=False),
        scratch_types=[
            pltpu.VMEM((TB, D), f32),
            pltpu.VMEM((TB, D), f32),
            pltpu.VMEM((TB, D), f32),
            pltpu.VMEM((TB, D), f32),
            pltpu.VMEM((HC // 128, 128), jnp.int32),
            pltpu.VMEM((HC,), jnp.int32),
            pltpu.VMEM((T * D,), f32),
            pltpu.VMEM((T * D,), f32),
            pltpu.VMEM((T * T,), f32),
            pltpu.VMEM((T,), f32),
            pltpu.VMEM((D * D,), f32),
            pltpu.VMEM((D,), f32),
            pltpu.VMEM((128,), f32),
            pltpu.VMEM((B,), jnp.int32),
            pltpu.SemaphoreType.DMA,
            pltpu.SemaphoreType.DMA,
            pltpu.SemaphoreType.DMA,
            pltpu.SemaphoreType.DMA,
            pltpu.VMEM_SHARED((16, T * D), f32),
            pltpu.VMEM_SHARED((8, T * D), f32),
            pltpu.VMEM_SHARED((8, 128), f32),
        ],
    )(_body)
    x2 = x.reshape(L * B, D)
    irT = jnp.transpose(ir, (1, 0))
    out2 = kfn(x2, irT, vl, Wh.reshape(T * T), bh, Ww.reshape(D * D), bw)
    return out2.reshape(L, B, D)


def kernel(sequence_data, is_rnn, v_len, Wh, bh, Ww, bw):
    seq_out = _fwd(sequence_data, is_rnn.astype(jnp.int32),
                   v_len.astype(jnp.int32), Wh, bh, Ww, bw)
    return (seq_out, is_rnn)


# speedup vs baseline: 11.5813x; 1.1771x over previous
import functools
import jax
import jax.numpy as jnp
from jax import lax
from jax.experimental import pallas as pl
from jax.experimental.pallas import tpu as pltpu
from jax.experimental.pallas import tpu_sc as plsc

L, B, D, T = 4096, 16, 64, 32
HC = L // 2
TB = 256
NSUB = HC // TB
NG = TB // 16
NQ = TB // 128
NT = NSUB // 2


def _body(x_hbm, ir_hbm, vl_hbm, wh_hbm, bh_hbm, ww_hbm, bw_hbm, out_hbm,
          ib0, ib1, ob0, ob1, idxc, mbuf, attv, pbuf,
          whb, bhb, wwb, bwb, hwb, vlb,
          gsm0, gsm1, ssm0, ssm1, shatt, shf32, shhw):
    c = lax.axis_index("c")
    s = lax.axis_index("s")
    si = s // 2
    ch = s % 2
    i = c * 8 + si
    f32 = jnp.float32

    pltpu.sync_copy(vl_hbm, vlb)
    pltpu.sync_copy(ir_hbm.at[i, pl.ds(ch * HC, HC)], mbuf)

    iota16b = lax.iota(jnp.int32, 16) * B

    def _bld(q, _):
        for v in range(8):
            basei = (ch * HC + q * 128 + v * 16) * B + i
            idxc[q, pl.ds(v * 16, 16)] = iota16b + basei
        return 0

    lax.fori_loop(0, HC // 128, _bld, 0)

    ibufs = (ib0, ib1)
    obufs = (ob0, ob1)
    gsems = (gsm0, gsm1)
    ssems = (ssm0, ssm1)

    def issue_gather(sb, buf, sem):
        for qq in range(NQ):
            pltpu.async_copy(x_hbm.at[idxc.at[sb * NQ + qq]],
                             buf.at[pl.ds(qq * 128, 128)], sem)

    def wait_gather(sb, buf, sem):
        for qq in range(NQ):
            pltpu.make_async_copy(x_hbm.at[idxc.at[sb * NQ + qq]],
                                  buf.at[pl.ds(qq * 128, 128)], sem).wait()

    def issue_scatter(sb, buf, sem):
        for qq in range(NQ):
            pltpu.async_copy(buf.at[pl.ds(qq * 128, 128)],
                             out_hbm.at[idxc.at[sb * NQ + qq]], sem)

    def wait_scatter(sb, buf, sem):
        for qq in range(NQ):
            pltpu.make_async_copy(buf.at[pl.ds(qq * 128, 128)],
                                  out_hbm.at[idxc.at[sb * NQ + qq]],
                                  sem).wait()

    mv0 = vlb[pl.ds(0, 16)]
    m = jnp.sum(jnp.where(lax.iota(jnp.int32, 16) == i, mv0, 0))

    zeros16i = jnp.zeros((16,), jnp.int32)

    def _cnt(j, acc):
        return acc + mbuf[pl.ds(j * 16, 16)]

    cvec = lax.fori_loop(0, HC // 16, _cnt, zeros16i)
    my_cnt = jnp.sum(cvec)
    prefix = jnp.where(ch == 0, 0, m - my_cnt)

    big = jnp.maximum(m, T)
    small = jnp.clip(jnp.minimum(m, T), 1, T)
    gs = big // small
    rem = big % small
    thr = rem * (gs + 1)
    mmax = jnp.maximum(m - 1, 0)

    def bucket(p):
        bb = jnp.where(p < thr, p // (gs + 1), rem + (p - thr) // gs)
        return jnp.clip(bb, 0, T - 1)

    zeros16f = jnp.zeros((16,), f32)

    def _z(j, _):
        attv[pl.ds(j * 16, 16)] = zeros16f
        pbuf[pl.ds(j * 16, 16)] = zeros16f
        return 0

    lax.fori_loop(0, (T * D) // 16, _z, 0)

    is_lt = m < T

    @pl.when((ch == 0) & is_lt)
    def _zero_first32():
        pltpu.sync_copy(pbuf, shf32.at[si])

    @pl.when(ch == 0)
    def _stage_w():
        pltpu.sync_copy(wh_hbm, whb)
        pltpu.sync_copy(bh_hbm, bhb)
        pltpu.sync_copy(ww_hbm, wwb)
        pltpu.sync_copy(bw_hbm, bwb)

    plsc.subcore_barrier()

    def _p1_compute(sb, carry_in, xcur):
        def grp(gi, carry):
            cnt, cb, a0, a1, a2, a3 = carry
            base = sb * TB + gi * 16
            mvec = mbuf[pl.ds(base, 16)]
            cums = plsc.cumsum(mvec)
            pvec = jnp.clip(prefix + cnt + cums - 1, 0, mmax)
            bvec = bucket(pvec)
            mff = mvec.astype(f32)
            uniform = (bvec[0] == cb) & (bvec[15] == cb)

            def _fast(ops):
                cb0, b0, b1, b2, b3 = ops
                accs = [b0, b1, b2, b3]
                for j in range(16):
                    mj = mff[j]
                    for g in range(4):
                        accs[g] = (accs[g]
                                   + xcur[gi * 16 + j,
                                          pl.ds(g * 16, 16)] * mj)
                return (cb0, accs[0], accs[1], accs[2], accs[3])

            def _slow(ops):
                cb0, b0, b1, b2, b3 = ops
                for g, bg in enumerate((b0, b1, b2, b3)):
                    off = cb0 * D + g * 16
                    attv[pl.ds(off, 16)] = attv[pl.ds(off, 16)] + bg
                for j in range(16):
                    bj = bvec[j]
                    mj = mff[j]
                    for g in range(4):
                        off = bj * D + g * 16
                        attv[pl.ds(off, 16)] = (
                            attv[pl.ds(off, 16)]
                            + xcur[gi * 16 + j, pl.ds(g * 16, 16)] * mj)
                return (bvec[15], zeros16f, zeros16f, zeros16f, zeros16f)

            cb, a0, a1, a2, a3 = lax.cond(uniform, _fast, _slow,
                                          (cb, a0, a1, a2, a3))
            return (cnt + cums[15], cb, a0, a1, a2, a3)

        carry_out = lax.fori_loop(0, NG, grp, carry_in)
        cnt_in = carry_in[0]

        @pl.when(is_lt)
        def _first32():
            def grp2(gi, cnt):
                base = sb * TB + gi * 16
                mvec = mbuf[pl.ds(base, 16)]
                cums = plsc.cumsum(mvec)
                pvec = prefix + cnt + cums - 1
                for j in range(16):
                    pj = pvec[j]

                    @pl.when((mvec[j] > 0) & (pj < T))
                    def _():
                        pltpu.sync_copy(xcur.at[gi * 16 + j],
                                        shf32.at[si, pl.ds(pj * D, D)])

                return cnt + cums[15]

            lax.fori_loop(0, NG, grp2, cnt_in)

        return carry_out

    issue_gather(0, ibufs[0], gsems[0])
    issue_gather(1, ibufs[1], gsems[1])

    def _p1_outer(t2, carry):
        for b2 in range(2):
            sb = 2 * t2 + b2
            wait_gather(sb, ibufs[b2], gsems[b2])
            carry = _p1_compute(sb, carry, ibufs[b2])

            @pl.when(t2 < NT - 1)
            def _():
                issue_gather(sb + 2, ibufs[b2], gsems[b2])

        return carry

    fin = lax.fori_loop(0, NT, _p1_outer,
                        (jnp.int32(0), jnp.int32(0),
                         zeros16f, zeros16f, zeros16f, zeros16f))
    cbf = fin[1]
    for g in range(4):
        offf = cbf * D + g * 16
        attv[pl.ds(offf, 16)] = attv[pl.ds(offf, 16)] + fin[2 + g]

    pltpu.sync_copy(attv, shatt.at[s])
    plsc.subcore_barrier()

    @pl.when(ch == 0)
    def _phase2():
        pltpu.sync_copy(shatt.at[s + 1], pbuf)

        def addp(j, _):
            attv[pl.ds(j * 16, 16)] = (attv[pl.ds(j * 16, 16)]
                                       + pbuf[pl.ds(j * 16, 16)])
            return 0

        lax.fori_loop(0, (T * D) // 16, addp, 0)

        @pl.when(m >= T)
        def _gt_mean():
            def rowdiv(j, _):
                cj = (gs + jnp.where(j < rem, 1, 0)).astype(f32)
                rcv = jnp.full((16,), 1.0, f32) / cj
                for g in range(4):
                    off = j * D + g * 16
                    attv[pl.ds(off, 16)] = attv[pl.ds(off, 16)] * rcv
                return 0

            lax.fori_loop(0, T, rowdiv, 0)

        @pl.when(is_lt)
        def _lt_expand():
            pltpu.sync_copy(shf32.at[si], pbuf)

            def rowcp(j, _):
                sidx = jnp.where(j < thr, j // (gs + 1),
                                 rem + (j - thr) // gs)
                sidx = jnp.clip(sidx, 0, T - 1)
                for g in range(4):
                    attv[pl.ds(j * D + g * 16, 16)] = (
                        pbuf[pl.ds(sidx * D + g * 16, 16)])
                return 0

            lax.fori_loop(0, T, rowcp, 0)

        iota16 = lax.iota(jnp.int32, 16)

        ad = [zeros16f, zeros16f]
        for j in range(T):
            acc = attv[pl.ds(j * D, 16)]
            for g in range(1, 4):
                acc = acc + attv[pl.ds(j * D + g * 16, 16)]
            rsum = jnp.sum(acc) * (1.0 / D)
            gslot = j // 16
            ad[gslot] = ad[gslot] + jnp.where(iota16 == (j % 16), rsum, 0.0)

        def colacc(j, carry):
            return tuple(carry[g] + attv[pl.ds(j * D + g * 16, 16)]
                         for g in range(4))

        at4 = [v * (1.0 / T) for v in
               lax.fori_loop(0, T, colacc, (zeros16f,) * 4)]

        hv = []
        for g in range(2):
            zacc = zeros16f
            for u in range(T):
                col = plsc.load_gather(whb, [iota16 * T + (g * 16 * T + u)])
                zacc = zacc + col * ad[u // 16][u % 16]
            z = zacc + bhb[pl.ds(g * 16, 16)]
            hv.append(1.0 / (1.0 + jnp.exp(-z)))

        for g in range(4):
            zacc = zeros16f
            for e in range(D):
                col = plsc.load_gather(wwb, [iota16 * D + (g * 16 * D + e)])
                zacc = zacc + col * at4[e // 16][e % 16]
            z = zacc + bwb[pl.ds(g * 16, 16)]
            hwb[pl.ds(T + g * 16, 16)] = 1.0 / (1.0 + jnp.exp(-z))

        @pl.when(m >= T)
        def _hc_gt():
            for g in range(2):
                hwb[pl.ds(g * 16, 16)] = hv[g]

        @pl.when(is_lt)
        def _hc_lt():
            hacc = [zeros16f, zeros16f]
            hcnt = [zeros16f, zeros16f]
            for j in range(T):
                sidx = jnp.where(j < thr, j // (gs + 1),
                                 rem + (j - thr) // gs)
                sidx = jnp.clip(sidx, 0, T - 1)
                hj = hv[j // 16][j % 16]
                for g in range(2):
                    hit = (iota16 + g * 16) == sidx
                    hacc[g] = hacc[g] + jnp.where(hit, hj, 0.0)
                    hcnt[g] = hcnt[g] + jnp.where(hit, 1.0, 0.0)
            for g in range(2):
                hwb[pl.ds(g * 16, 16)] = (hacc[g]
                                          / jnp.maximum(hcnt[g], 1.0))

        pltpu.sync_copy(hwb, shhw.at[si])

    plsc.subcore_barrier()
    pltpu.sync_copy(shhw.at[si], hwb)

    whalf = [hwb[pl.ds(T + g * 16, 16)] * 0.5 for g in range(4)]
    m_ge = m >= T

    def _p3_compute(sb, cnt_in, xcur, xout):
        def grp(gi, cnt):
            base = sb * TB + gi * 16
            mvec = mbuf[pl.ds(base, 16)]
            cums = plsc.cumsum(mvec)
            pvec = jnp.clip(prefix + cnt + cums - 1, 0, mmax)
            idxv = jnp.where(m_ge, bucket(pvec), pvec)
            hs2 = plsc.load_gather(hwb, [idxv]) * 0.5
            mff = mvec.astype(f32)
            omv = 1.0 - mff
            for j in range(16):
                hj = hs2[j]
                mj = mff[j]
                oj = omv[j]
                for g in range(4):
                    scale = (whalf[g] + hj) * mj + oj
                    xout[gi * 16 + j, pl.ds(g * 16, 16)] = (
                        xcur[gi * 16 + j, pl.ds(g * 16, 16)] * scale)
            return cnt + cums[15]

        return lax.fori_loop(0, NG, grp, cnt_in)

    issue_gather(0, ibufs[0], gsems[0])
    issue_gather(1, ibufs[1], gsems[1])

    def _p3_outer(t2, cnt):
        for b2 in range(2):
            sb = 2 * t2 + b2
            wait_gather(sb, ibufs[b2], gsems[b2])

            @pl.when(t2 > 0)
            def _():
                wait_scatter(sb - 2, obufs[b2], ssems[b2])

            cnt = _p3_compute(sb, cnt, ibufs[b2], obufs[b2])
            issue_scatter(sb, obufs[b2], ssems[b2])

            @pl.when(t2 < NT - 1)
            def _():
                issue_gather(sb + 2, ibufs[b2], gsems[b2])

        return cnt

    lax.fori_loop(0, NT, _p3_outer, jnp.int32(0))
    for b2 in range(2):
        wait_scatter(NSUB - 2 + b2, obufs[b2], ssems[b2])


@jax.jit
def _fwd(x, ir, vl, Wh, bh, Ww, bw):
    mesh = plsc.VectorSubcoreMesh(core_axis_name="c", subcore_axis_name="s")
    f32 = jnp.float32
    kfn = functools.partial(
        pl.kernel,
        out_type=jax.ShapeDtypeStruct((L * B, D), f32),
        mesh=mesh,
        compiler_params=pltpu.CompilerParams(use_tc_tiling_on_sc=False,
                                             needs_layout_passes=False),
        scratch_types=[
            pltpu.VMEM((TB, D), f32),
            pltpu.VMEM((TB, D), f32),
            pltpu.VMEM((TB, D), f32),
            pltpu.VMEM((TB, D), f32),
            pltpu.VMEM((HC // 128, 128), jnp.int32),
            pltpu.VMEM((HC,), jnp.int32),
            pltpu.VMEM((T * D,), f32),
            pltpu.VMEM((T * D,), f32),
            pltpu.VMEM((T * T,), f32),
            pltpu.VMEM((T,), f32),
            pltpu.VMEM((D * D,), f32),
            pltpu.VMEM((D,), f32),
            pltpu.VMEM((128,), f32),
            pltpu.VMEM((B,), jnp.int32),
            pltpu.SemaphoreType.DMA,
            pltpu.SemaphoreType.DMA,
            pltpu.SemaphoreType.DMA,
            pltpu.SemaphoreType.DMA,
            pltpu.VMEM_SHARED((16, T * D), f32),
            pltpu.VMEM_SHARED((8, T * D), f32),
            pltpu.VMEM_SHARED((8, 128), f32),
        ],
    )(_body)
    x2 = x.reshape(L * B, D)
    irT = jnp.transpose(ir, (1, 0))
    out2 = kfn(x2, irT, vl, Wh.reshape(T * T), bh, Ww.reshape(D * D), bw)
    return out2.reshape(L, B, D)


def kernel(sequence_data, is_rnn, v_len, Wh, bh, Ww, bw):
    seq_out = _fwd(sequence_data, is_rnn.astype(jnp.int32),
                   v_len.astype(jnp.int32), Wh, bh, Ww, bw)
    return (seq_out, is_rnn)
